# Initial kernel scaffold; baseline (speedup 1.0000x reference)
import jax
import jax.numpy as jnp
from jax.experimental import pallas as pl


def kernel(x, E_class):
    raise NotImplementedError("write your pallas kernel here")


# speedup vs baseline: 23.1414x; 23.1414x over previous
import functools

import jax
import jax.numpy as jnp
from jax import lax
from jax.experimental import pallas as pl
from jax.experimental.pallas import tpu as pltpu
from jax.experimental.pallas import tpu_sc as plsc

B, S, CLASS_NUM, E_DIMS, LEVELS = 4096, 50, 100000, 256, 128
N_TOK = B * S
CLASS_COLS = tuple(range(0, S, 8))
N_CLS = len(CLASS_COLS)

_T_OVER_PI = float((2.0 ** 30) / 3.14159265358979323846)

_C1 = 3.141592653589793
_C3 = -5.16771278004997
_C5 = 2.550164039877345
_C7 = -0.5992645293207921
_C9 = 0.08214588661112823

_TOK_BLK = 512


def _enc_body(x_ref, o_ref):
    v = x_ref[...]
    t = v * 4097.0
    a = t - (t - v)
    b = v - a

    lane = lax.broadcasted_iota(jnp.int32, (_TOK_BLK, 256), 1)
    k = lane >> 1
    parity = lane & 1
    pk = lax.bitcast_convert_type((k + 127) << 23, jnp.float32)

    wa = a * pk
    ra = wa - 2.0 * jnp.floor(wa * 0.5)
    wb = b * pk
    rb = wb - 2.0 * jnp.floor(wb * 0.5)
    r = ra + rb
    r = jnp.where(r >= 2.0, r - 2.0, r)

    r2 = r + 0.5 * parity.astype(jnp.float32)
    q = jnp.floor(r2 + 0.5)
    s = r2 - q
    s2 = s * s
    res = ((((_C9 * s2 + _C7) * s2 + _C5) * s2 + _C3) * s2 + _C1) * s
    qh = q * 0.5
    odd = (qh - jnp.floor(qh)) > 0.25
    res = jnp.where(odd, -res, res)

    thr = jnp.where(k == 127, jnp.float32(-1.0), jnp.float32(_T_OVER_PI))
    res = jnp.where(v * pk > thr, jnp.float32(jnp.nan), res)
    o_ref[...] = res


_enc_call = pl.pallas_call(
    _enc_body,
    grid=(N_TOK // _TOK_BLK,),
    in_specs=[pl.BlockSpec((_TOK_BLK, 1), lambda i: (i, jnp.int32(0)))],
    out_specs=pl.BlockSpec((_TOK_BLK, 256), lambda i: (i, jnp.int32(0))),
    out_shape=jax.ShapeDtypeStruct((N_TOK, 256), jnp.float32),
    compiler_params=pltpu.CompilerParams(
        dimension_semantics=("arbitrary",),
    ),
)


def _sc_body(out_ref, x_ref, e_ref, xv, idx_v, oidx_v, rows_v, sem_g, sem_s):
    nc = 2
    wid = lax.axis_index("s") * nc + lax.axis_index("c")
    rows_per_w = B // 32
    base = wid * rows_per_w

    pltpu.sync_copy(x_ref.at[pl.ds(base * S, rows_per_w * S)], xv)

    lanes = lax.broadcasted_iota(jnp.int32, (16,), 0)

    def chunk(c, buf):
        for g in range(rows_per_w // 16):
            rows16 = lanes + 16 * g
            vals = plsc.load_gather(xv, [rows16 * S + 8 * c])
            idx_v[buf][pl.ds(16 * g, 16)] = vals.astype(jnp.int32)
            oidx_v[buf][pl.ds(16 * g, 16)] = (base + rows16) * S + 8 * c

    for c in range(N_CLS):
        buf = c % 2
        chunk(c, buf)
        pltpu.async_copy(e_ref.at[idx_v[buf]], rows_v[buf], sem_g).wait()
        pltpu.async_copy(rows_v[buf], out_ref.at[oidx_v[buf]], sem_s).wait()


@functools.cache
def _get_sc_call():
    return pl.kernel(
        _sc_body,
        out_type=(),
        mesh=plsc.VectorSubcoreMesh(core_axis_name="c", subcore_axis_name="s"),
        compiler_params=pltpu.CompilerParams(needs_layout_passes=False),
        scratch_types=dict(
            xv=pltpu.VMEM((B // 32 * S,), jnp.float32),
            idx_v=[pltpu.VMEM((B // 32,), jnp.int32) for _ in range(2)],
            oidx_v=[pltpu.VMEM((B // 32,), jnp.int32) for _ in range(2)],
            rows_v=[pltpu.VMEM((B // 32, E_DIMS), jnp.float32)
                    for _ in range(2)],
            sem_g=pltpu.SemaphoreType.DMA,
            sem_s=pltpu.SemaphoreType.DMA,
        ),
    )


def kernel(x, E_class):
    enc = _enc_call(x.reshape(N_TOK, 1))
    out = jax.new_ref(enc)
    _get_sc_call()(out, x.reshape(N_TOK), E_class)
    return out[...].reshape(B, S, 256)


# speedup vs baseline: 25.1222x; 1.0856x over previous
import functools

import jax
import jax.numpy as jnp
from jax import lax
from jax.experimental import pallas as pl
from jax.experimental.pallas import tpu as pltpu
from jax.experimental.pallas import tpu_sc as plsc

B, S, CLASS_NUM, E_DIMS, LEVELS = 4096, 50, 100000, 256, 128
N_TOK = B * S
N_CLS = (S + 7) // 8

_T_OVER_PI = float((2.0 ** 30) / 3.14159265358979323846)

_C1 = 3.141592653589793
_C3 = -5.16771278004997
_C5 = 2.550164039877345
_C7 = -0.5992645293207921
_C9 = 0.08214588661112823

_ROWS_BLK = 16
_TOK_BLK = _ROWS_BLK * S


def _enc_body(x_ref, g_ref, o_ref):
    v = x_ref[...]
    t = v * 4097.0
    a = t - (t - v)
    b = v - a

    lane = lax.broadcasted_iota(jnp.int32, (_TOK_BLK, 256), 1)
    k = lane >> 1
    parity = lane & 1
    pk = lax.bitcast_convert_type((k + 127) << 23, jnp.float32)

    wa = a * pk
    ra = wa - 2.0 * jnp.floor(wa * 0.5)
    wb = b * pk
    rb = wb - 2.0 * jnp.floor(wb * 0.5)
    r = ra + rb
    r = jnp.where(r >= 2.0, r - 2.0, r)

    r2 = r + 0.5 * parity.astype(jnp.float32)
    q = jnp.floor(r2 + 0.5)
    s = r2 - q
    s2 = s * s
    res = ((((_C9 * s2 + _C7) * s2 + _C5) * s2 + _C3) * s2 + _C1) * s
    qh = q * 0.5
    odd = (qh - jnp.floor(qh)) > 0.25
    res = jnp.where(odd, -res, res)

    thr = jnp.where(k == 127, jnp.float32(-1.0), jnp.float32(_T_OVER_PI))
    res = jnp.where(v * pk > thr, jnp.float32(jnp.nan), res)
    o_ref[...] = res

    for j in range(_ROWS_BLK):
        for c in range(N_CLS):
            o_ref[pl.ds(S * j + 8 * c, 1), :] = g_ref[c, pl.ds(j, 1), :]


_enc_call = pl.pallas_call(
    _enc_body,
    grid=(N_TOK // _TOK_BLK,),
    in_specs=[
        pl.BlockSpec((_TOK_BLK, 1), lambda i: (i, jnp.int32(0))),
        pl.BlockSpec((N_CLS, _ROWS_BLK, 256),
                     lambda i: (jnp.int32(0), i, jnp.int32(0))),
    ],
    out_specs=pl.BlockSpec((_TOK_BLK, 256), lambda i: (i, jnp.int32(0))),
    out_shape=jax.ShapeDtypeStruct((N_TOK, 256), jnp.float32),
    compiler_params=pltpu.CompilerParams(
        dimension_semantics=("arbitrary",),
    ),
)


def _sc_body(x_ref, e_ref, g_ref, xv, idx_v, rows_v, sem_g, sem_s):
    nc = 2
    wid = lax.axis_index("s") * nc + lax.axis_index("c")
    rows_per_w = B // 32
    base = wid * rows_per_w

    pltpu.sync_copy(x_ref.at[pl.ds(base * S, rows_per_w * S)], xv)

    lanes = lax.broadcasted_iota(jnp.int32, (16,), 0)

    def chunk(c, buf):
        for g in range(rows_per_w // 16):
            rows16 = lanes + 16 * g
            vals = plsc.load_gather(xv, [rows16 * S + 8 * c])
            idx_v[buf][pl.ds(16 * g, 16)] = vals.astype(jnp.int32)

    stores = [None, None]
    for c in range(N_CLS):
        buf = c % 2
        if stores[buf] is not None:
            stores[buf].wait()
        chunk(c, buf)
        pltpu.async_copy(e_ref.at[idx_v[buf]], rows_v[buf], sem_g).wait()
        stores[buf] = pltpu.async_copy(
            rows_v[buf], g_ref.at[jnp.int32(c), pl.ds(base, rows_per_w)],
            sem_s)
    for st in stores:
        if st is not None:
            st.wait()


@functools.cache
def _get_sc_call():
    return pl.kernel(
        _sc_body,
        out_type=jax.ShapeDtypeStruct((N_CLS, B, E_DIMS), jnp.float32),
        mesh=plsc.VectorSubcoreMesh(core_axis_name="c", subcore_axis_name="s"),
        compiler_params=pltpu.CompilerParams(needs_layout_passes=False),
        scratch_types=dict(
            xv=pltpu.VMEM((B // 32 * S,), jnp.float32),
            idx_v=[pltpu.VMEM((B // 32,), jnp.int32) for _ in range(2)],
            rows_v=[pltpu.VMEM((B // 32, E_DIMS), jnp.float32)
                    for _ in range(2)],
            sem_g=pltpu.SemaphoreType.DMA,
            sem_s=pltpu.SemaphoreType.DMA,
        ),
    )


def kernel(x, E_class):
    x_flat = x.reshape(N_TOK)
    g = _get_sc_call()(x_flat, E_class)
    out = _enc_call(x_flat.reshape(N_TOK, 1), g)
    return out.reshape(B, S, 256)
